# baseline (speedup 1.0000x reference)
import functools

import jax
import jax.numpy as jnp
from jax import lax
from jax.experimental import pallas as pl
from jax.experimental.pallas import tpu as pltpu
from jax.experimental.pallas import tpu_sc as plsc

BATCH_N = 16384
DIM_N = 16
CH = 128
PACK = 128 // DIM_N


def _make_sc_kernel():
    info = plsc.get_sparse_core_info()
    NC, NS = info.num_cores, info.num_subcores
    NW = NC * NS
    b_per_w = BATCH_N // NW
    n_chunks = b_per_w // CH
    out_rows_w = b_per_w // PACK
    mesh = plsc.VectorSubcoreMesh(core_axis_name="c", subcore_axis_name="s")

    @functools.partial(
        pl.kernel,
        mesh=mesh,
        out_type=jax.ShapeDtypeStruct((BATCH_N // PACK, 128), jnp.float32),
        compiler_params=pltpu.CompilerParams(use_tc_tiling_on_sc=True),
        scratch_types=[
            pltpu.VMEM((n_chunks, CH), jnp.int32),
            pltpu.VMEM((n_chunks, CH), jnp.int32),
            pltpu.VMEM((n_chunks, CH), jnp.int32),
            pltpu.VMEM((n_chunks, CH), jnp.int32),
            pltpu.VMEM((CH, 128), jnp.float32),
            pltpu.VMEM((CH, 128), jnp.float32),
            pltpu.VMEM((CH, 128), jnp.float32),
            pltpu.VMEM((CH, 128), jnp.float32),
            pltpu.VMEM((out_rows_w, 128), jnp.float32),
            pltpu.SemaphoreType.DMA,
            pltpu.SemaphoreType.DMA,
        ],
    )
    def k(user_hbm, item_hbm, uemb_hbm, iemb_hbm, out_hbm,
          uidx_v, iidx_v, urow_v, irow_v, ubuf0, ubuf1, ibuf0, ibuf1,
          outbuf, sem0, sem1):
        wid = lax.axis_index("s") * NC + lax.axis_index("c")
        pltpu.sync_copy(user_hbm.at[pl.ds(wid * n_chunks, n_chunks)], uidx_v)
        pltpu.sync_copy(item_hbm.at[pl.ds(wid * n_chunks, n_chunks)], iidx_v)
        for j in range(n_chunks):
            for t in range(CH // 16):
                sl = pl.ds(t * 16, 16)
                urow_v[j, sl] = lax.shift_right_logical(uidx_v[j, sl], 3)
                irow_v[j, sl] = lax.shift_right_logical(iidx_v[j, sl], 3)

        sems = (sem0, sem1)
        ubufs = (ubuf0, ubuf1)
        ibufs = (ibuf0, ibuf1)

        def start(c):
            s = sems[c % 2]
            return (pltpu.async_copy(uemb_hbm.at[urow_v.at[c]], ubufs[c % 2], s),
                    pltpu.async_copy(iemb_hbm.at[irow_v.at[c]], ibufs[c % 2], s))

        pending = {0: start(0)}
        for c in range(n_chunks):
            if c + 1 < n_chunks:
                pending[c + 1] = start(c + 1)
            for d in pending.pop(c):
                d.wait()
            ub = ubufs[c % 2]
            ib = ibufs[c % 2]

            def body(g, carry, c=c, ub=ub, ib=ib):
                su_vec = uidx_v[c, pl.ds(g * 16, 16)]
                si_vec = iidx_v[c, pl.ds(g * 16, 16)]
                for j in range(16):
                    ou = lax.mul(lax.bitwise_and(su_vec[j], 7), 16)
                    oi = lax.mul(lax.bitwise_and(si_vec[j], 7), 16)
                    r = g * 16 + j
                    hu = ub[r, pl.ds(ou, 16)]
                    hi = ib[r, pl.ds(oi, 16)]
                    orow = c * (CH // PACK) + g * 2 + (j // 8)
                    outbuf[orow, pl.ds((j % 8) * 16, 16)] = hu * hi
                return carry

            lax.fori_loop(0, CH // 16, body, 0)
        pltpu.sync_copy(outbuf, out_hbm.at[pl.ds(wid * out_rows_w, out_rows_w)])

    return k


def kernel(user, item, user_emb, item_emb):
    k = _make_sc_kernel()
    user2 = user.reshape(BATCH_N // CH, CH)
    item2 = item.reshape(BATCH_N // CH, CH)
    uemb2 = user_emb.reshape(-1, 128)
    iemb2 = item_emb.reshape(-1, 128)
    out = k(user2, item2, uemb2, iemb2)
    return out.reshape(BATCH_N, DIM_N)


# speedup vs baseline: 5.8665x; 5.8665x over previous
import functools

import jax
import jax.numpy as jnp
from jax import lax
from jax.experimental import pallas as pl
from jax.experimental.pallas import tpu as pltpu
from jax.experimental.pallas import tpu_sc as plsc

BATCH_N = 16384
DIM_N = 16
G = 8
COL = 128


def _make_sc_kernel():
    info = plsc.get_sparse_core_info()
    NC, NS = info.num_cores, info.num_subcores
    NW = NC * NS
    b_per_w = BATCH_N // NW
    n_batches = b_per_w // G
    mesh = plsc.VectorSubcoreMesh(core_axis_name="c", subcore_axis_name="s")

    @functools.partial(
        pl.kernel,
        mesh=mesh,
        out_type=jax.ShapeDtypeStruct((DIM_N, BATCH_N), jnp.float32),
        compiler_params=pltpu.CompilerParams(use_tc_tiling_on_sc=True,
                                             needs_layout_passes=False),
        scratch_types=[
            pltpu.VMEM((b_per_w + 16,), jnp.int32),
            pltpu.VMEM((b_per_w + 16,), jnp.int32),
            pltpu.VMEM((DIM_N, G * COL), jnp.float32),
            pltpu.VMEM((DIM_N, G * COL), jnp.float32),
            pltpu.VMEM((DIM_N, G * COL), jnp.float32),
            pltpu.VMEM((DIM_N, G * COL), jnp.float32),
            pltpu.VMEM((DIM_N, b_per_w), jnp.float32),
            pltpu.SemaphoreType.DMA,
            pltpu.SemaphoreType.DMA,
        ],
    )
    def k(user_hbm, item_hbm, uemb_hbm, iemb_hbm, out_hbm,
          flat_u, flat_i, ubufA, ubufB, ibufA, ibufB, outbuf, semA, semB):
        wid = lax.axis_index("s") * NC + lax.axis_index("c")
        nch = b_per_w // 128
        for c in range(nch):
            pltpu.sync_copy(user_hbm.at[wid * nch + c],
                            flat_u.at[pl.ds(c * 128, 128)])
            pltpu.sync_copy(item_hbm.at[wid * nch + c],
                            flat_i.at[pl.ds(c * 128, 128)])
        flat_u[pl.ds(b_per_w, 16)] = jnp.zeros((16,), jnp.int32)
        flat_i[pl.ds(b_per_w, 16)] = jnp.zeros((16,), jnp.int32)

        di = lax.iota(jnp.int32, 16)

        def idx_vecs(g):
            vu = flat_u[pl.ds(lax.mul(g, G), 16)]
            vi = flat_i[pl.ds(lax.mul(g, G), 16)]
            return vu, vi

        def fire(g, ubuf, ibuf, sem):
            vu, vi = idx_vecs(g)
            u0 = lax.bitwise_and(vu, -COL)
            i0 = lax.bitwise_and(vi, -COL)
            for j in range(G):
                dst = pl.ds(j * COL, COL)
                pltpu.async_copy(
                    uemb_hbm.at[:, pl.ds(pl.multiple_of(u0[j], COL), COL)],
                    ubuf.at[:, dst], sem)
                pltpu.async_copy(
                    iemb_hbm.at[:, pl.ds(pl.multiple_of(i0[j], COL), COL)],
                    ibuf.at[:, dst], sem)

        def compute(g, ubuf, ibuf):
            vu, vi = idx_vecs(g)
            cu = lax.bitwise_and(vu, COL - 1)
            ci = lax.bitwise_and(vi, COL - 1)
            for j in range(G):
                hu = plsc.load_gather(
                    ubuf, [di, jnp.full((16,), cu[j] + j * COL, jnp.int32)])
                hi = plsc.load_gather(
                    ibuf, [di, jnp.full((16,), ci[j] + j * COL, jnp.int32)])
                b_local = lax.add(lax.mul(g, G), j)
                plsc.store_scatter(
                    outbuf, [di, jnp.full((16,), b_local, jnp.int32)], hu * hi)

        def drain(sem, ubuf, ibuf):
            hbm_dummy = uemb_hbm.at[:, pl.ds(0, G * COL)]
            pltpu.make_async_copy(hbm_dummy, ubuf, sem).wait()
            pltpu.make_async_copy(hbm_dummy, ibuf, sem).wait()

        fire(jnp.int32(0), ubufA, ibufA, semA)
        fire(jnp.int32(1), ubufB, ibufB, semB)

        def body(gp, carry):
            g0 = lax.mul(gp, 2)
            drain(semA, ubufA, ibufA)
            compute(g0, ubufA, ibufA)
            fire(lax.add(g0, 2), ubufA, ibufA, semA)
            drain(semB, ubufB, ibufB)
            compute(lax.add(g0, 1), ubufB, ibufB)
            fire(lax.add(g0, 3), ubufB, ibufB, semB)
            return carry

        lax.fori_loop(0, n_batches // 2 - 1, body, 0)
        drain(semA, ubufA, ibufA)
        compute(jnp.int32(n_batches - 2), ubufA, ibufA)
        drain(semB, ubufB, ibufB)
        compute(jnp.int32(n_batches - 1), ubufB, ibufB)

        pltpu.sync_copy(outbuf, out_hbm.at[:, pl.ds(wid * b_per_w, b_per_w)])

    return k


def kernel(user, item, user_emb, item_emb):
    k = _make_sc_kernel()
    user2 = user.reshape(BATCH_N // 128, 128)
    item2 = item.reshape(BATCH_N // 128, 128)
    out_t = k(user2, item2, user_emb.T, item_emb.T)
    return out_t.T
